# Initial kernel scaffold; baseline (speedup 1.0000x reference)
import jax
import jax.numpy as jnp
from jax.experimental import pallas as pl


def kernel(x, edge_index, W1, b1, W2, b2):
    raise NotImplementedError("write your pallas kernel here")


# speedup vs baseline: 3.2938x; 3.2938x over previous
import functools

import jax
import jax.numpy as jnp
from jax import lax
from jax.experimental import pallas as pl
from jax.experimental.pallas import tpu as pltpu
from jax.experimental.pallas import tpu_sc as plsc

N = 10000
E = 160000
D = 256
H = 1024
O = 256

NC = 2
NS = 16
L = 16

DH = D // NC
EPT = E // NS
CHUNK = 80
NCHUNK = EPT // CHUNK
NP = 10240
RPT = NP // NS
RCH = 128
NRCH = RPT // RCH


def _agg_body(x2_hbm, src_hbm, dst_hbm, out_hbm,
              src_v, dst_v, gidx_v, rows_v, buf_v, agg_sh, sem):
    c = lax.axis_index("c")
    s = lax.axis_index("s")

    def zero_body(i, _):
        r = i // (DH // L)
        col = (i % (DH // L)) * L
        buf_v[r, pl.ds(col, L)] = jnp.zeros((L,), jnp.float32)
        return 0
    lax.fori_loop(0, RCH * (DH // L), zero_body, 0)
    for t in range(NRCH):
        pltpu.sync_copy(buf_v, agg_sh.at[pl.ds(s * RPT + t * RCH, RCH)])
    plsc.subcore_barrier()

    base_e = s * EPT

    def edge_body(j, _):
        e0 = base_e + j * CHUNK
        pltpu.sync_copy(src_hbm.at[pl.ds(e0, CHUNK)], src_v)
        pltpu.sync_copy(dst_hbm.at[pl.ds(e0, CHUNK)], dst_v)
        for k in range(CHUNK // L):
            sl = pl.ds(k * L, L)
            gidx_v[sl] = src_v[sl] * 2 + c
        pltpu.async_copy(x2_hbm.at[gidx_v], rows_v, sem).wait()
        pltpu.sync_copy(rows_v, agg_sh.at[dst_v], add=True)
        return 0

    lax.fori_loop(0, NCHUNK, edge_body, 0)
    plsc.subcore_barrier()

    for t in range(NRCH):
        r0 = s * RPT + t * RCH
        pltpu.sync_copy(agg_sh.at[pl.ds(r0, RCH)], buf_v)
        pltpu.sync_copy(buf_v, out_hbm.at[c].at[pl.ds(r0, RCH)])


@functools.cache
def _make_agg_kernel():
    return pl.kernel(
        _agg_body,
        out_type=jax.ShapeDtypeStruct((NC, NP, DH), jnp.float32),
        mesh=plsc.VectorSubcoreMesh(
            core_axis_name="c", subcore_axis_name="s", num_cores=NC,
            num_subcores=NS),
        scratch_types=[
            pltpu.VMEM((CHUNK,), jnp.int32),
            pltpu.VMEM((CHUNK,), jnp.int32),
            pltpu.VMEM((CHUNK,), jnp.int32),
            pltpu.VMEM((CHUNK, DH), jnp.float32),
            pltpu.VMEM((RCH, DH), jnp.float32),
            pltpu.VMEM_SHARED((NP, DH), jnp.float32),
            pltpu.SemaphoreType.DMA,
        ],
    )


BN = 1000


def _mlp_body(a_ref, w1_ref, b1_ref, w2_ref, b2_ref, o_ref):
    h = jnp.dot(a_ref[0], w1_ref[:DH, :], preferred_element_type=jnp.float32)
    h = h + jnp.dot(a_ref[1], w1_ref[DH:, :],
                    preferred_element_type=jnp.float32)
    h = jnp.maximum(h + b1_ref[...], 0.0)
    o_ref[...] = (jnp.dot(h, w2_ref[...], preferred_element_type=jnp.float32)
                  + b2_ref[...])


def _mlp(agg2, W1, b1, W2, b2):
    return pl.pallas_call(
        _mlp_body,
        grid=(N // BN,),
        in_specs=[
            pl.BlockSpec((NC, BN, DH), lambda i: (0, i, 0)),
            pl.BlockSpec((D, H), lambda i: (0, 0)),
            pl.BlockSpec((1, H), lambda i: (0, 0)),
            pl.BlockSpec((H, O), lambda i: (0, 0)),
            pl.BlockSpec((1, O), lambda i: (0, 0)),
        ],
        out_specs=pl.BlockSpec((BN, O), lambda i: (i, 0)),
        out_shape=jax.ShapeDtypeStruct((N, O), jnp.float32),
    )(agg2, W1, b1, W2, b2)


def kernel(x, edge_index, W1, b1, W2, b2):
    x2 = x.reshape(2 * N, DH)
    agg2 = _make_agg_kernel()(x2, edge_index[0], edge_index[1])
    return _mlp(agg2, W1, b1.reshape(1, H), W2, b2.reshape(1, O))


# speedup vs baseline: 5.9017x; 1.7918x over previous
import functools

import jax
import jax.numpy as jnp
from jax import lax
from jax.experimental import pallas as pl
from jax.experimental.pallas import tpu as pltpu
from jax.experimental.pallas import tpu_sc as plsc

N = 10000
E = 160000
D = 256
H = 1024
O = 256

NC = 2
NS = 16
L = 16

DH = D // NC
EPT = E // NS
CHUNK = 80
NCHUNK = EPT // CHUNK
NP = 10240
RPT = NP // NS
RCH = CHUNK
NRCH = RPT // RCH


def _agg_body(x2_hbm, src_hbm, dst_hbm, out_hbm,
              src0_v, src1_v, dst0_v, dst1_v, rows_v, agg_sh,
              isem0, isem1, gsem0, gsem1):
    c = lax.axis_index("c")
    s = lax.axis_index("s")
    buf_v = rows_v.at[0]

    def zero_body(i, _):
        r = i // (DH // L)
        col = (i % (DH // L)) * L
        buf_v[r, pl.ds(col, L)] = jnp.zeros((L,), jnp.float32)
        return 0
    lax.fori_loop(0, RCH * (DH // L), zero_body, 0)
    for t in range(NRCH):
        pltpu.sync_copy(buf_v, agg_sh.at[pl.ds(s * RPT + t * RCH, RCH)])

    plsc.subcore_barrier()

    base_e = s * EPT
    srcs = (src0_v, src1_v)
    dsts = (dst0_v, dst1_v)
    isems = (isem0, isem1)
    gsems = (gsem0, gsem1)

    def issue_idx(jb, b):
        e0 = base_e + jb * CHUNK
        pltpu.async_copy(src_hbm.at[pl.ds(e0, CHUNK)], srcs[b], isems[b])
        pltpu.async_copy(dst_hbm.at[pl.ds(e0, CHUNK)], dsts[b], isems[b])

    def fire_gather(jb, b):
        e0 = base_e + jb * CHUNK
        pltpu.make_async_copy(
            src_hbm.at[pl.ds(e0, CHUNK)], srcs[b], isems[b]).wait()
        pltpu.make_async_copy(
            dst_hbm.at[pl.ds(e0, CHUNK)], dsts[b], isems[b]).wait()
        for k in range(CHUNK // L):
            sl = pl.ds(k * L, L)
            srcs[b][sl] = srcs[b][sl] * 2 + c
        pltpu.async_copy(x2_hbm.at[srcs[b]], rows_v.at[b], gsems[b])

    def finish(jb, b):
        pltpu.make_async_copy(
            x2_hbm.at[srcs[b]], rows_v.at[b], gsems[b]).wait()
        pltpu.sync_copy(rows_v.at[b], agg_sh.at[dsts[b]], add=True)

    issue_idx(0, 0)
    fire_gather(0, 0)
    issue_idx(1, 1)

    def pipe_body(i, _):
        j0 = 2 * i
        fire_gather(j0 + 1, 1)
        finish(j0, 0)

        @pl.when(j0 + 2 < NCHUNK)
        def _():
            issue_idx(j0 + 2, 0)

        finish(j0 + 1, 1)

        @pl.when(j0 + 2 < NCHUNK)
        def _():
            fire_gather(j0 + 2, 0)

        @pl.when(j0 + 3 < NCHUNK)
        def _():
            issue_idx(j0 + 3, 1)

        return 0

    lax.fori_loop(0, (NCHUNK - 1) // 2, pipe_body, 0)
    finish(NCHUNK - 1, 0)
    plsc.subcore_barrier()

    for t in range(NRCH):
        r0 = s * RPT + t * RCH
        pltpu.sync_copy(agg_sh.at[pl.ds(r0, RCH)], buf_v)
        pltpu.sync_copy(buf_v, out_hbm.at[c].at[pl.ds(r0, RCH)])


@functools.cache
def _make_agg_kernel():
    return pl.kernel(
        _agg_body,
        out_type=jax.ShapeDtypeStruct((NC, NP, DH), jnp.float32),
        mesh=plsc.VectorSubcoreMesh(
            core_axis_name="c", subcore_axis_name="s", num_cores=NC,
            num_subcores=NS),
        scratch_types=[
            pltpu.VMEM((CHUNK,), jnp.int32),
            pltpu.VMEM((CHUNK,), jnp.int32),
            pltpu.VMEM((CHUNK,), jnp.int32),
            pltpu.VMEM((CHUNK,), jnp.int32),
            pltpu.VMEM((2, CHUNK, DH), jnp.float32),
            pltpu.VMEM_SHARED((NP, DH), jnp.float32),
            pltpu.SemaphoreType.DMA,
            pltpu.SemaphoreType.DMA,
            pltpu.SemaphoreType.DMA,
            pltpu.SemaphoreType.DMA,
        ],
    )


BN = 1000


def _mlp_body(a_ref, w1_ref, b1_ref, w2_ref, b2_ref, o_ref):
    h = jnp.dot(a_ref[0], w1_ref[:DH, :], preferred_element_type=jnp.float32)
    h = h + jnp.dot(a_ref[1], w1_ref[DH:, :],
                    preferred_element_type=jnp.float32)
    h = jnp.maximum(h + b1_ref[...], 0.0)
    o_ref[...] = (jnp.dot(h, w2_ref[...], preferred_element_type=jnp.float32)
                  + b2_ref[...])


def _mlp(agg2, W1, b1, W2, b2):
    return pl.pallas_call(
        _mlp_body,
        grid=(N // BN,),
        in_specs=[
            pl.BlockSpec((NC, BN, DH), lambda i: (0, i, 0)),
            pl.BlockSpec((D, H), lambda i: (0, 0)),
            pl.BlockSpec((1, H), lambda i: (0, 0)),
            pl.BlockSpec((H, O), lambda i: (0, 0)),
            pl.BlockSpec((1, O), lambda i: (0, 0)),
        ],
        out_specs=pl.BlockSpec((BN, O), lambda i: (i, 0)),
        out_shape=jax.ShapeDtypeStruct((N, O), jnp.float32),
    )(agg2, W1, b1, W2, b2)


def kernel(x, edge_index, W1, b1, W2, b2):
    x2 = x.reshape(2 * N, DH)
    agg2 = _make_agg_kernel()(x2, edge_index[0], edge_index[1])
    return _mlp(agg2, W1, b1.reshape(1, H), W2, b2.reshape(1, O))


# speedup vs baseline: 7.0277x; 1.1908x over previous
import functools

import jax
import jax.numpy as jnp
from jax import lax
from jax.experimental import pallas as pl
from jax.experimental.pallas import tpu as pltpu
from jax.experimental.pallas import tpu_sc as plsc

N = 10000
E = 160000
D = 256
H = 1024
O = 256

NC = 2
NS = 16
L = 16

DH = D // NC
EPT = E // NS
CHUNK = 80
NCHUNK = EPT // CHUNK
NP = 10240
RPT = NP // NS
RCH = CHUNK
NRCH = RPT // RCH


NB = 4


def _agg_body(x2_hbm, src_hbm, dst_hbm, out_hbm,
              src0_v, src1_v, src2_v, src3_v,
              dst0_v, dst1_v, dst2_v, dst3_v,
              dss0_v, dss1_v, dss2_v, dss3_v,
              rows_v, agg_sh,
              isem0, isem1, isem2, isem3,
              gsem0, gsem1, gsem2, gsem3,
              ssem0, ssem1, ssem2, ssem3):
    c = lax.axis_index("c")
    s = lax.axis_index("s")
    buf_v = rows_v.at[0]

    def zero_body(i, _):
        r = i // (DH // L)
        col = (i % (DH // L)) * L
        buf_v[r, pl.ds(col, L)] = jnp.zeros((L,), jnp.float32)
        return 0
    lax.fori_loop(0, RCH * (DH // L), zero_body, 0)
    for t in range(NRCH):
        pltpu.sync_copy(buf_v, agg_sh.at[pl.ds(s * RPT + t * RCH, RCH)])

    plsc.subcore_barrier()

    base_e = s * EPT
    srcs = (src0_v, src1_v, src2_v, src3_v)
    dsts = (dst0_v, dst1_v, dst2_v, dst3_v)
    dsss = (dss0_v, dss1_v, dss2_v, dss3_v)
    isems = (isem0, isem1, isem2, isem3)
    gsems = (gsem0, gsem1, gsem2, gsem3)
    ssems = (ssem0, ssem1, ssem2, ssem3)

    def issue_idx(jb, u):
        e0 = base_e + jb * CHUNK
        pltpu.async_copy(src_hbm.at[pl.ds(e0, CHUNK)], srcs[u], isems[u])
        pltpu.async_copy(dst_hbm.at[pl.ds(e0, CHUNK)], dsts[u], isems[u])

    def drain_scatter(u):
        pltpu.make_async_copy(
            rows_v.at[u], agg_sh.at[dsss[u]], ssems[u]).wait()

    def fire(jb, u):
        e0 = base_e + jb * CHUNK
        pltpu.make_async_copy(
            src_hbm.at[pl.ds(e0, CHUNK)], srcs[u], isems[u]).wait()
        pltpu.make_async_copy(
            dst_hbm.at[pl.ds(e0, CHUNK)], dsts[u], isems[u]).wait()
        for k in range(CHUNK // L):
            sl = pl.ds(k * L, L)
            dsss[u][sl] = dsts[u][sl]
            srcs[u][sl] = srcs[u][sl] * 2 + c
        pltpu.async_copy(x2_hbm.at[srcs[u]], rows_v.at[u], gsems[u])

    def finish(u):
        pltpu.make_async_copy(
            x2_hbm.at[srcs[u]], rows_v.at[u], gsems[u]).wait()
        pltpu.async_copy(rows_v.at[u], agg_sh.at[dsss[u]], ssems[u],
                         add=True)

    issue_idx(0, 0)
    issue_idx(1, 1)
    issue_idx(2, 2)
    fire(0, 0)
    issue_idx(3, 3)

    def pipe_body(i, _):
        jg = NB * i
        for uu in range(1, NB + 1):
            jb = jg + uu
            u = uu % NB
            if uu == NB:
                drain_scatter(u)
            else:
                @pl.when(i >= 1)
                def _(u=u):
                    drain_scatter(u)
            fire(jb, u)
            finish((u + NB - 1) % NB)

            @pl.when(jb + 3 < NCHUNK)
            def _(jb=jb, u=u):
                issue_idx(jb + 3, (u + 3) % NB)

        return 0

    lax.fori_loop(0, (NCHUNK - 1) // NB, pipe_body, 0)
    finish((NCHUNK - 1) % NB)
    for u in range(NB):
        drain_scatter(u)
    plsc.subcore_barrier()

    for t in range(NRCH):
        r0 = s * RPT + t * RCH
        pltpu.sync_copy(agg_sh.at[pl.ds(r0, RCH)], buf_v)
        pltpu.sync_copy(buf_v, out_hbm.at[c].at[pl.ds(r0, RCH)])


@functools.cache
def _make_agg_kernel():
    return pl.kernel(
        _agg_body,
        out_type=jax.ShapeDtypeStruct((NC, NP, DH), jnp.float32),
        mesh=plsc.VectorSubcoreMesh(
            core_axis_name="c", subcore_axis_name="s", num_cores=NC,
            num_subcores=NS),
        scratch_types=(
            [pltpu.VMEM((CHUNK,), jnp.int32)] * (3 * NB)
            + [pltpu.VMEM((NB, CHUNK, DH), jnp.float32),
               pltpu.VMEM_SHARED((NP, DH), jnp.float32)]
            + [pltpu.SemaphoreType.DMA] * (3 * NB)
        ),
    )


BN = 1000


def _mlp_body(a_ref, w1_ref, b1_ref, w2_ref, b2_ref, o_ref):
    h = jnp.dot(a_ref[0], w1_ref[:DH, :], preferred_element_type=jnp.float32)
    h = h + jnp.dot(a_ref[1], w1_ref[DH:, :],
                    preferred_element_type=jnp.float32)
    h = jnp.maximum(h + b1_ref[...], 0.0)
    o_ref[...] = (jnp.dot(h, w2_ref[...], preferred_element_type=jnp.float32)
                  + b2_ref[...])


def _mlp(agg2, W1, b1, W2, b2):
    return pl.pallas_call(
        _mlp_body,
        grid=(N // BN,),
        in_specs=[
            pl.BlockSpec((NC, BN, DH), lambda i: (0, i, 0)),
            pl.BlockSpec((D, H), lambda i: (0, 0)),
            pl.BlockSpec((1, H), lambda i: (0, 0)),
            pl.BlockSpec((H, O), lambda i: (0, 0)),
            pl.BlockSpec((1, O), lambda i: (0, 0)),
        ],
        out_specs=pl.BlockSpec((BN, O), lambda i: (i, 0)),
        out_shape=jax.ShapeDtypeStruct((N, O), jnp.float32),
    )(agg2, W1, b1, W2, b2)


def kernel(x, edge_index, W1, b1, W2, b2):
    x2 = x.reshape(2 * N, DH)
    agg2 = _make_agg_kernel()(x2, edge_index[0], edge_index[1])
    return _mlp(agg2, W1, b1.reshape(1, H), W2, b2.reshape(1, O))
